# baseline (speedup 1.0000x reference)
import functools

import jax
import jax.numpy as jnp
from jax import lax
from jax.experimental import pallas as pl
from jax.experimental.pallas import tpu as pltpu
from jax.experimental.pallas import tpu_sc as plsc

_NC = 2
_NS = 16
_L = 16
_VOCAB = 100000


def _sc_body(q_hbm, k_hbm, v_hbm, out_hbm, tab, qv, kv, vv, ov):
    B, S2 = q_hbm.shape
    nchunks = S2 // (2 * _L)
    rows_per_w = B // (_NC * _NS)
    wid = lax.axis_index("s") * _NC + lax.axis_index("c")
    lane = lax.iota(jnp.int32, _L)
    lane2 = lane * jnp.int32(2)
    zero16 = jnp.zeros((_L,), jnp.int32)
    l2L = jnp.int32(2 * _L)

    for r in range(rows_per_w):
        row = wid * jnp.int32(rows_per_w) + jnp.int32(r)
        pltpu.sync_copy(q_hbm.at[row], qv)
        pltpu.sync_copy(k_hbm.at[row], kv)
        pltpu.sync_copy(v_hbm.at[row], vv)

        def zero_body(c, carry):
            base2 = c * l2L
            plsc.store_scatter(tab, [plsc.load_gather(qv, [base2 + lane2])],
                               zero16)
            plsc.store_scatter(tab, [plsc.load_gather(kv, [base2 + lane2])],
                               zero16)
            return carry

        lax.fori_loop(jnp.int32(0), jnp.int32(nchunks), zero_body,
                      jnp.int32(0), unroll=False)

        def chunk_body(c, carry):
            base2 = c * l2L
            qc = plsc.load_gather(qv, [base2 + lane2])
            kc = plsc.load_gather(kv, [base2 + lane2])
            vc = plsc.load_gather(vv, [base2 + lane2])
            outc = plsc.load_gather(tab, [qc])
            dup = qc != qc
            for j in range(_L):
                idxj = jnp.full((_L,), base2 + jnp.int32(2 * j), jnp.int32)
                kj = plsc.load_gather(kv, [idxj])
                vj = plsc.load_gather(vv, [idxj])
                outc = jnp.where((qc == kj) & (lane > j), vj, outc)
                dup = dup | ((kc == kj) & (lane < j))
            plsc.store_scatter(tab, [kc], vc, mask=jnp.logical_not(dup))
            plsc.store_scatter(ov, [base2 + lane2], outc)
            plsc.store_scatter(ov, [base2 + lane2 + jnp.int32(1)], zero16)
            return carry

        lax.fori_loop(jnp.int32(0), jnp.int32(nchunks), chunk_body,
                      jnp.int32(0), unroll=False)
        pltpu.sync_copy(ov, out_hbm.at[row])


def kernel(q, k, v):
    B, S = q.shape
    qx = lax.bitcast_convert_type(q, jnp.int32).reshape(B, 2 * S)
    kx = lax.bitcast_convert_type(k, jnp.int32).reshape(B, 2 * S)
    vx = lax.bitcast_convert_type(v, jnp.int32).reshape(B, 2 * S)

    mesh = plsc.VectorSubcoreMesh(core_axis_name="c", subcore_axis_name="s")
    run = functools.partial(
        pl.kernel,
        out_type=jax.ShapeDtypeStruct((B, 2 * S), jnp.int32),
        mesh=mesh,
        scratch_types=[
            pltpu.VMEM((_VOCAB,), jnp.int32),
            pltpu.VMEM((2 * S,), jnp.int32),
            pltpu.VMEM((2 * S,), jnp.int32),
            pltpu.VMEM((2 * S,), jnp.int32),
            pltpu.VMEM((2 * S,), jnp.int32),
        ],
        compiler_params=pltpu.CompilerParams(needs_layout_passes=False),
    )(_sc_body)
    out = run(qx, kx, vx)
    return lax.bitcast_convert_type(out.reshape(B, S, 2), jnp.int64)


# speedup vs baseline: 1.3024x; 1.3024x over previous
import functools

import jax
import jax.numpy as jnp
from jax import lax
from jax.experimental import pallas as pl
from jax.experimental.pallas import tpu as pltpu
from jax.experimental.pallas import tpu_sc as plsc

_NC = 2
_NS = 16
_L = 16
_VOCAB = 100000
_VSHIFT = 17
_VMASK = (1 << _VSHIFT) - 1


def _treemax(xs):
    while len(xs) > 1:
        nxt = [jnp.maximum(xs[i], xs[i + 1]) for i in range(0, len(xs) - 1, 2)]
        if len(xs) % 2:
            nxt.append(xs[-1])
        xs = nxt
    return xs[0]


def _treeor(xs):
    while len(xs) > 1:
        nxt = [xs[i] | xs[i + 1] for i in range(0, len(xs) - 1, 2)]
        if len(xs) % 2:
            nxt.append(xs[-1])
        xs = nxt
    return xs[0]


_GDN = lax.GatherDimensionNumbers(
    offset_dims=(), collapsed_slice_dims=(0,), start_index_map=(0,))


def _bcast(x, j):
    idx = jnp.full((_L, 1), j, jnp.int32)
    return lax.gather(x, idx, _GDN, (1,),
                      mode=lax.GatherScatterMode.PROMISE_IN_BOUNDS)


def _sc_body(q_hbm, k_hbm, v_hbm, out_hbm, tab, qv, kv, vv, pv, ov):
    B, S = q_hbm.shape
    nchunks = S // _L
    rows_per_w = B // (_NC * _NS)
    wid = lax.axis_index("s") * _NC + lax.axis_index("c")
    lane = lax.iota(jnp.int32, _L)
    zero16 = jnp.zeros((_L,), jnp.int32)
    packtag = (lane + jnp.int32(1)) << _VSHIFT
    lL = jnp.int32(_L)

    for r in range(rows_per_w):
        row = wid * jnp.int32(rows_per_w) + jnp.int32(r)
        pltpu.sync_copy(q_hbm.at[row], qv)
        pltpu.sync_copy(k_hbm.at[row], kv)
        pltpu.sync_copy(v_hbm.at[row], vv)

        def zero_body(c, carry):
            base = c * lL
            sl = pl.ds(base, _L)
            plsc.store_scatter(tab, [qv[sl]], zero16)
            plsc.store_scatter(tab, [kv[sl]], zero16)
            pv[sl] = vv[sl] | packtag
            return carry

        lax.fori_loop(jnp.int32(0), jnp.int32(nchunks), zero_body,
                      jnp.int32(0), unroll=False)

        def chunk_body(c, carry):
            base = c * lL
            sl = pl.ds(base, _L)
            qc = qv[sl]
            kc = kv[sl]
            vc = vv[sl]
            pc = pv[sl]
            tabres = plsc.load_gather(tab, [qc])
            cands = []
            dups = []
            for j in range(_L):
                kj = _bcast(kc, j)
                if j < _L - 1:
                    pj = _bcast(pc, j)
                    cands.append(
                        jnp.where((qc == kj) & (lane > j), pj, zero16))
                if j > 0:
                    dups.append((kc == kj) & (lane < j))
            best = _treemax(cands)
            dup = _treeor(dups)
            plsc.store_scatter(tab, [kc], vc, mask=jnp.logical_not(dup))
            outc = jnp.where(best > jnp.int32(0), best & jnp.int32(_VMASK),
                             tabres)
            ov[sl] = outc
            return carry

        lax.fori_loop(jnp.int32(0), jnp.int32(nchunks), chunk_body,
                      jnp.int32(0), unroll=False)
        pltpu.sync_copy(ov, out_hbm.at[row])


def kernel(q, k, v):
    B, S = q.shape
    q32 = q.astype(jnp.int32)
    k32 = k.astype(jnp.int32)
    v32 = v.astype(jnp.int32)

    mesh = plsc.VectorSubcoreMesh(core_axis_name="c", subcore_axis_name="s")
    run = functools.partial(
        pl.kernel,
        out_type=jax.ShapeDtypeStruct((B, S), jnp.int32),
        mesh=mesh,
        scratch_types=[
            pltpu.VMEM((_VOCAB,), jnp.int32),
            pltpu.VMEM((S,), jnp.int32),
            pltpu.VMEM((S,), jnp.int32),
            pltpu.VMEM((S,), jnp.int32),
            pltpu.VMEM((S,), jnp.int32),
            pltpu.VMEM((S,), jnp.int32),
        ],
        compiler_params=pltpu.CompilerParams(needs_layout_passes=False),
    )(_sc_body)
    out = run(q32, k32, v32)
    return out.astype(q.dtype)


# speedup vs baseline: 1.3233x; 1.0160x over previous
import functools

import jax
import jax.numpy as jnp
from jax import lax
from jax.experimental import pallas as pl
from jax.experimental.pallas import tpu as pltpu
from jax.experimental.pallas import tpu_sc as plsc

_NC = 2
_NS = 16
_L = 16
_VOCAB = 100000
_VSHIFT = 17
_VMASK = (1 << _VSHIFT) - 1


def _treemax(xs):
    while len(xs) > 1:
        nxt = [jnp.maximum(xs[i], xs[i + 1]) for i in range(0, len(xs) - 1, 2)]
        if len(xs) % 2:
            nxt.append(xs[-1])
        xs = nxt
    return xs[0]


def _treeor(xs):
    while len(xs) > 1:
        nxt = [xs[i] | xs[i + 1] for i in range(0, len(xs) - 1, 2)]
        if len(xs) % 2:
            nxt.append(xs[-1])
        xs = nxt
    return xs[0]


_GDN = lax.GatherDimensionNumbers(
    offset_dims=(), collapsed_slice_dims=(0,), start_index_map=(0,))


def _bcast(x, j):
    idx = jnp.full((_L, 1), j, jnp.int32)
    return lax.gather(x, idx, _GDN, (1,),
                      mode=lax.GatherScatterMode.PROMISE_IN_BOUNDS)


def _sc_body(x_hbm, out_hbm, tab, xv, pv, ov):
    B, S3 = x_hbm.shape
    S = S3 // 3
    nchunks = S // _L
    rows_per_w = B // (_NC * _NS)
    wid = lax.axis_index("s") * _NC + lax.axis_index("c")
    lane = lax.iota(jnp.int32, _L)
    zero16 = jnp.zeros((_L,), jnp.int32)
    packtag = (lane + jnp.int32(1)) << _VSHIFT
    lL = jnp.int32(_L)
    sS = jnp.int32(S)

    for r in range(rows_per_w):
        row = wid * jnp.int32(rows_per_w) + jnp.int32(r)
        pltpu.sync_copy(x_hbm.at[row], xv)

        def zero_body(c, carry):
            for u in range(4):
                base = c * jnp.int32(4 * _L) + jnp.int32(u * _L)
                plsc.store_scatter(tab, [xv[pl.ds(base, _L)]], zero16)
                plsc.store_scatter(tab, [xv[pl.ds(base + sS, _L)]], zero16)
                pv[pl.ds(base, _L)] = (
                    xv[pl.ds(base + jnp.int32(2 * S), _L)] | packtag)
            return carry

        lax.fori_loop(jnp.int32(0), jnp.int32(nchunks // 4), zero_body,
                      jnp.int32(0), unroll=False)

        def chunk_body(c, carry):
            base = c * lL
            qc = xv[pl.ds(base, _L)]
            kc = xv[pl.ds(base + sS, _L)]
            vc = xv[pl.ds(base + jnp.int32(2 * S), _L)]
            pc = pv[pl.ds(base, _L)]
            tabres = plsc.load_gather(tab, [qc])
            cands = []
            dups = []
            for j in range(_L):
                kj = _bcast(kc, j)
                if j < _L - 1:
                    pj = _bcast(pc, j)
                    cands.append(
                        jnp.where((qc == kj) & (lane > j), pj, zero16))
                if j > 0:
                    dups.append((kc == kj) & (lane < j))
            best = _treemax(cands)
            dup = _treeor(dups)
            plsc.store_scatter(tab, [kc], vc, mask=jnp.logical_not(dup))
            outc = jnp.where(best > jnp.int32(0), best & jnp.int32(_VMASK),
                             tabres)
            ov[pl.ds(base, _L)] = outc
            return carry

        lax.fori_loop(jnp.int32(0), jnp.int32(nchunks), chunk_body,
                      jnp.int32(0), unroll=False)
        pltpu.sync_copy(ov, out_hbm.at[row])


def kernel(q, k, v):
    B, S = q.shape
    x = jnp.concatenate([q, k, v], axis=1).astype(jnp.int32)

    mesh = plsc.VectorSubcoreMesh(core_axis_name="c", subcore_axis_name="s")
    run = functools.partial(
        pl.kernel,
        out_type=jax.ShapeDtypeStruct((B, S), jnp.int32),
        mesh=mesh,
        scratch_types=[
            pltpu.VMEM((_VOCAB,), jnp.int32),
            pltpu.VMEM((3 * S,), jnp.int32),
            pltpu.VMEM((S,), jnp.int32),
            pltpu.VMEM((S,), jnp.int32),
        ],
        compiler_params=pltpu.CompilerParams(needs_layout_passes=False),
    )(_sc_body)
    out = run(x)
    return out.astype(q.dtype)


# speedup vs baseline: 1.9943x; 1.5071x over previous
import jax
import jax.numpy as jnp
from jax import lax
from jax.experimental import pallas as pl

_VSHIFT = 17
_VMASK = (1 << _VSHIFT) - 1
_BB = 8
_TT = 64


def _tc_body(q_ref, k_ref, p_ref, o_ref):
    S = q_ref.shape[1]
    nt = S // _TT
    for bi in range(_BB):
        qrow = q_ref[bi:bi + 1, :]
        kcol = k_ref[bi:bi + 1, :].reshape(S, 1)
        pcol = p_ref[bi:bi + 1, :].reshape(S, 1)
        acc = jnp.zeros((1, S), jnp.int32)
        for ti in range(nt):
            kt = kcol[ti * _TT:(ti + 1) * _TT]
            pt = pcol[ti * _TT:(ti + 1) * _TT]
            tp = ti * _TT + lax.broadcasted_iota(jnp.int32, (_TT, S), 0)
            t = lax.broadcasted_iota(jnp.int32, (_TT, S), 1)
            hit = (kt == qrow) & (tp < t)
            part = jnp.max(jnp.where(hit, pt, 0), axis=0, keepdims=True)
            acc = jnp.maximum(acc, part)
        o_ref[bi:bi + 1, :] = acc & _VMASK


def kernel(q, k, v):
    B, S = q.shape
    q32 = q.astype(jnp.int32)
    k32 = k.astype(jnp.int32)
    packed = ((jnp.arange(S, dtype=jnp.int32) + 1) << _VSHIFT) | v.astype(
        jnp.int32)

    out = pl.pallas_call(
        _tc_body,
        grid=(B // _BB,),
        in_specs=[
            pl.BlockSpec((_BB, S), lambda b: (b, b * 0)),
            pl.BlockSpec((_BB, S), lambda b: (b, b * 0)),
            pl.BlockSpec((_BB, S), lambda b: (b, b * 0)),
        ],
        out_specs=pl.BlockSpec((_BB, S), lambda b: (b, b * 0)),
        out_shape=jax.ShapeDtypeStruct((B, S), jnp.int32),
    )(q32, k32, packed)
    return out.astype(q.dtype)
